# Initial kernel scaffold; baseline (speedup 1.0000x reference)
import jax
import jax.numpy as jnp
from jax.experimental import pallas as pl


def kernel(idx, targets, table):
    raise NotImplementedError("write your pallas kernel here")


# speedup vs baseline: 1.3666x; 1.3666x over previous
import functools

import jax
import jax.numpy as jnp
from jax import lax
from jax.experimental import pallas as pl
from jax.experimental.pallas import tpu as pltpu, tpu_sc as plsc

VOCAB = 1000
B, T = 1024, 50
N_TOK = B * T
LSE_PAD = 1024

NC, NS = 2, 16
NW = NC * NS
TOK_PER_W = N_TOK // NW
CHUNK = 32
N_CHUNKS = TOK_PER_W // CHUNK
L = 16


def _row_lse_body(table_ref, out_ref):
    t = table_ref[...]
    m = jnp.max(t, axis=1, keepdims=True)
    s = jnp.sum(jnp.exp(t - m), axis=1, keepdims=True)
    out_ref[0:VOCAB, :] = m + jnp.log(s)


def _row_lse(table):
    out = pl.pallas_call(
        _row_lse_body,
        out_shape=jax.ShapeDtypeStruct((LSE_PAD, 1), jnp.float32),
    )(table)
    return out.reshape(LSE_PAD)


def _sc_body(idx_hbm, tgt_hbm, lse_hbm, table_hbm, out_hbm, part_hbm,
             idx_v, tgt_v, lse_v, rows_v, acc_v, sem):
    cid = lax.axis_index("c")
    sid = lax.axis_index("s")
    wid = sid * NC + cid
    base = wid * TOK_PER_W

    pltpu.sync_copy(idx_hbm.at[pl.ds(base, TOK_PER_W)], idx_v)
    pltpu.sync_copy(tgt_hbm.at[pl.ds(base, TOK_PER_W)], tgt_v)
    pltpu.sync_copy(lse_hbm, lse_v)

    def step(g, acc):
        off = g * CHUNK
        pltpu.async_copy(
            table_hbm.at[idx_v.at[pl.ds(off, CHUNK)]], rows_v, sem).wait()
        pltpu.sync_copy(rows_v, out_hbm.at[pl.ds(base + off, CHUNK)])
        for j in range(CHUNK // L):
            idxv = idx_v[pl.ds(off + j * L, L)]
            tgtv = tgt_v[pl.ds(off + j * L, L)]
            lsev = plsc.load_gather(lse_v, [idxv])
            rowids = lax.iota(jnp.int32, L) + (j * L)
            pick = plsc.load_gather(rows_v, [rowids, tgtv])
            acc = acc + (lsev - pick)
        return acc

    acc = lax.fori_loop(0, N_CHUNKS, step, jnp.zeros((L,), jnp.float32))
    acc_v[...] = acc
    pltpu.sync_copy(acc_v, part_hbm.at[wid])


@functools.partial(
    pl.kernel,
    out_type=(
        jax.ShapeDtypeStruct((N_TOK, VOCAB), jnp.float32),
        jax.ShapeDtypeStruct((NW, L), jnp.float32),
    ),
    mesh=plsc.VectorSubcoreMesh(core_axis_name="c", subcore_axis_name="s"),
    compiler_params=pltpu.CompilerParams(
        needs_layout_passes=False, use_tc_tiling_on_sc=False),
    scratch_types=[
        pltpu.VMEM((TOK_PER_W,), jnp.int32),
        pltpu.VMEM((TOK_PER_W,), jnp.int32),
        pltpu.VMEM((LSE_PAD,), jnp.float32),
        pltpu.VMEM((CHUNK, VOCAB), jnp.float32),
        pltpu.VMEM((L,), jnp.float32),
        pltpu.SemaphoreType.DMA,
    ],
)
def _sc_gather_loss(idx_hbm, tgt_hbm, lse_hbm, table_hbm, out_hbm, part_hbm,
                    idx_v, tgt_v, lse_v, rows_v, acc_v, sem):
    _sc_body(idx_hbm, tgt_hbm, lse_hbm, table_hbm, out_hbm, part_hbm,
             idx_v, tgt_v, lse_v, rows_v, acc_v, sem)


def kernel(idx, targets, table):
    idx_flat = idx.reshape(N_TOK).astype(jnp.int32)
    tgt_flat = targets.reshape(N_TOK).astype(jnp.int32)
    lse = _row_lse(table)
    logits_flat, parts = _sc_gather_loss(idx_flat, tgt_flat, lse, table)
    logits = logits_flat.reshape(B, T, VOCAB)
    loss = jnp.sum(parts) / jnp.float32(N_TOK)
    return (logits, loss)


# speedup vs baseline: 1.4220x; 1.0406x over previous
import functools

import jax
import jax.numpy as jnp
from jax import lax
from jax.experimental import pallas as pl
from jax.experimental.pallas import tpu as pltpu, tpu_sc as plsc

VOCAB = 1000
B, T = 1024, 50
N_TOK = B * T
LSE_PAD = 1024

NC, NS = 2, 16
NW = NC * NS
TOK_PER_W = N_TOK // NW
CHUNK = 32
N_CHUNKS = TOK_PER_W // CHUNK
NBUF = 2
L = 16


def _row_lse_body(table_ref, out_ref):
    t = table_ref[...]
    m = jnp.max(t, axis=1, keepdims=True)
    s = jnp.sum(jnp.exp(t - m), axis=1, keepdims=True)
    out_ref[0:VOCAB, :] = m + jnp.log(s)


def _row_lse(table):
    out = pl.pallas_call(
        _row_lse_body,
        out_shape=jax.ShapeDtypeStruct((LSE_PAD, 1), jnp.float32),
    )(table)
    return out.reshape(LSE_PAD)


def _sc_body(idx_hbm, tgt_hbm, lse_hbm, table_hbm, out_hbm, part_hbm,
             idx_v, tgt_v, lse_v, rows_v, acc_v, gsem, ssem):
    cid = lax.axis_index("c")
    sid = lax.axis_index("s")
    wid = sid * NC + cid
    base = wid * TOK_PER_W

    pltpu.sync_copy(idx_hbm.at[pl.ds(base, TOK_PER_W)], idx_v)
    pltpu.sync_copy(tgt_hbm.at[pl.ds(base, TOK_PER_W)], tgt_v)
    pltpu.sync_copy(lse_hbm, lse_v)

    def gather_desc(g, b):
        return pltpu.make_async_copy(
            table_hbm.at[idx_v.at[pl.ds(g * CHUNK, CHUNK)]],
            rows_v.at[b], gsem)

    def scatter_desc(g, b):
        return pltpu.make_async_copy(
            rows_v.at[b], out_hbm.at[pl.ds(base + g * CHUNK, CHUNK)], ssem)

    gather_desc(0, 0).start()

    def step(g, acc):
        b = lax.rem(g, NBUF)
        gather_desc(g, b).wait()

        @pl.when(g >= 1)
        def _():
            scatter_desc(g - 1, 1 - b).wait()

        scatter_desc(g, b).start()

        @pl.when(g + 1 < N_CHUNKS)
        def _():
            gather_desc(g + 1, 1 - b).start()

        rows_b = rows_v.at[b]
        for j in range(CHUNK // L):
            idxv = idx_v[pl.ds(g * CHUNK + j * L, L)]
            tgtv = tgt_v[pl.ds(g * CHUNK + j * L, L)]
            lsev = plsc.load_gather(lse_v, [idxv])
            rowids = lax.iota(jnp.int32, L) + (j * L)
            pick = plsc.load_gather(rows_b, [rowids, tgtv])
            acc = acc + (lsev - pick)
        return acc

    acc = lax.fori_loop(0, N_CHUNKS, step, jnp.zeros((L,), jnp.float32))
    scatter_desc(N_CHUNKS - 1, (N_CHUNKS - 1) % NBUF).wait()
    acc_v[...] = acc
    pltpu.sync_copy(acc_v, part_hbm.at[wid])


@functools.partial(
    pl.kernel,
    out_type=(
        jax.ShapeDtypeStruct((N_TOK, VOCAB), jnp.float32),
        jax.ShapeDtypeStruct((NW, L), jnp.float32),
    ),
    mesh=plsc.VectorSubcoreMesh(core_axis_name="c", subcore_axis_name="s"),
    compiler_params=pltpu.CompilerParams(
        needs_layout_passes=False, use_tc_tiling_on_sc=False),
    scratch_types=[
        pltpu.VMEM((TOK_PER_W,), jnp.int32),
        pltpu.VMEM((TOK_PER_W,), jnp.int32),
        pltpu.VMEM((LSE_PAD,), jnp.float32),
        pltpu.VMEM((NBUF, CHUNK, VOCAB), jnp.float32),
        pltpu.VMEM((L,), jnp.float32),
        pltpu.SemaphoreType.DMA,
        pltpu.SemaphoreType.DMA,
    ],
)
def _sc_gather_loss(idx_hbm, tgt_hbm, lse_hbm, table_hbm, out_hbm, part_hbm,
                    idx_v, tgt_v, lse_v, rows_v, acc_v, gsem, ssem):
    _sc_body(idx_hbm, tgt_hbm, lse_hbm, table_hbm, out_hbm, part_hbm,
             idx_v, tgt_v, lse_v, rows_v, acc_v, gsem, ssem)


def kernel(idx, targets, table):
    idx_flat = idx.reshape(N_TOK).astype(jnp.int32)
    tgt_flat = targets.reshape(N_TOK).astype(jnp.int32)
    lse = _row_lse(table)
    logits_flat, parts = _sc_gather_loss(idx_flat, tgt_flat, lse, table)
    logits = logits_flat.reshape(B, T, VOCAB)
    loss = jnp.sum(parts) / jnp.float32(N_TOK)
    return (logits, loss)
